# baseline (speedup 1.0000x reference)
import functools

import jax
import jax.numpy as jnp
from jax import lax
from jax.experimental import pallas as pl
from jax.experimental.pallas import tpu as pltpu
from jax.experimental.pallas import tpu_sc as plsc

_PAD = 39972

_NC = 2
_NS = 16
_NW = _NC * _NS

_CH0 = 120
_CH1 = 80

_ROW_BLK = 1024


def _proj_body(tab_ref, w_ref, b_ref, out_ref):
    i = pl.program_id(0)
    row = i * _ROW_BLK + lax.broadcasted_iota(jnp.int32, (_ROW_BLK, 1), 0)
    t = jnp.where(row != _PAD, tab_ref[...], 0.0)
    y = jnp.dot(t, w_ref[...], preferred_element_type=jnp.float32)
    out_ref[...] = jnp.maximum(y + b_ref[...], 0.0)


def _project_table(table, W, b):
    V, D = table.shape
    H = W.shape[1]
    grid = pl.cdiv(V, _ROW_BLK)
    return pl.pallas_call(
        _proj_body,
        grid=(grid,),
        in_specs=[
            pl.BlockSpec((_ROW_BLK, D), lambda i: (i, 0)),
            pl.BlockSpec((D, H), lambda i: (0, 0)),
            pl.BlockSpec((1, H), lambda i: (0, 0)),
        ],
        out_specs=pl.BlockSpec((_ROW_BLK, H), lambda i: (i, 0)),
        out_shape=jax.ShapeDtypeStruct((V, H), jnp.float32),
    )(table, W, b.reshape(1, H))


def _make_gather(n, s, H):
    assert n % _NW == 0 and s == _CH0 + _CH1
    rpw = n // _NW
    ipw = rpw * s

    mesh = plsc.VectorSubcoreMesh(
        core_axis_name="c", subcore_axis_name="s",
        num_cores=_NC, num_subcores=_NS,
    )

    @functools.partial(
        pl.kernel,
        out_type=jax.ShapeDtypeStruct((n, s, H), jnp.float32),
        mesh=mesh,
        compiler_params=pltpu.CompilerParams(use_tc_tiling_on_sc=False),
        scratch_types=[
            pltpu.VMEM((ipw,), jnp.int32),
            pltpu.VMEM((_CH0, H), jnp.float32),
            pltpu.VMEM((_CH1, H), jnp.float32),
            pltpu.SemaphoreType.DMA,
            pltpu.SemaphoreType.DMA,
            pltpu.SemaphoreType.DMA,
            pltpu.SemaphoreType.DMA,
        ],
    )
    def gather(ytab_hbm, idx_hbm, out_hbm, idx_v, bufa, bufb, gsa, gsb, wsa, wsb):
        wid = lax.axis_index("s") * _NC + lax.axis_index("c")
        row0 = wid * rpw
        pltpu.sync_copy(idx_hbm.at[pl.ds(wid * ipw, ipw)], idx_v)

        def body(r, carry):
            @pl.when(r > 0)
            def _():
                pltpu.make_async_copy(
                    bufa, out_hbm.at[row0 + r - 1, pl.ds(0, _CH0)], wsa,
                ).wait()
                pltpu.make_async_copy(
                    bufb, out_hbm.at[row0 + r - 1, pl.ds(_CH0, _CH1)], wsb,
                ).wait()

            cpa = pltpu.async_copy(
                ytab_hbm.at[idx_v.at[pl.ds(r * s, _CH0)]], bufa, gsa)
            cpb = pltpu.async_copy(
                ytab_hbm.at[idx_v.at[pl.ds(r * s + _CH0, _CH1)]], bufb, gsb)
            cpa.wait()
            pltpu.async_copy(bufa, out_hbm.at[row0 + r, pl.ds(0, _CH0)], wsa)
            cpb.wait()
            pltpu.async_copy(bufb, out_hbm.at[row0 + r, pl.ds(_CH0, _CH1)], wsb)
            return carry

        lax.fori_loop(0, rpw, body, 0)
        pltpu.make_async_copy(
            bufa, out_hbm.at[row0 + rpw - 1, pl.ds(0, _CH0)], wsa).wait()
        pltpu.make_async_copy(
            bufb, out_hbm.at[row0 + rpw - 1, pl.ds(_CH0, _CH1)], wsb).wait()

    return gather


def kernel(pcp_batch, pcp_masks, table, W, b):
    n, s = pcp_batch.shape
    H = W.shape[1]
    ytab = _project_table(table, W, b)
    idx = pcp_batch.reshape(-1).astype(jnp.int32)
    y = _make_gather(n, s, H)(ytab, idx)
    return y, pcp_masks


# speedup vs baseline: 1.2132x; 1.2132x over previous
import functools

import jax
import jax.numpy as jnp
from jax import lax
from jax.experimental import pallas as pl
from jax.experimental.pallas import tpu as pltpu
from jax.experimental.pallas import tpu_sc as plsc

_PAD = 39972

_NC = 2
_NS = 16
_NW = _NC * _NS

_CH = (120, 80)
_OFF = (0, 120)

_ROW_BLK = 1024


def _proj_body(tab_ref, w_ref, b_ref, out_ref):
    i = pl.program_id(0)
    row = i * _ROW_BLK + lax.broadcasted_iota(jnp.int32, (_ROW_BLK, 1), 0)
    t = jnp.where(row != _PAD, tab_ref[...], 0.0)
    y = jnp.dot(t, w_ref[...], preferred_element_type=jnp.float32)
    out_ref[...] = jnp.maximum(y + b_ref[...], 0.0)


def _project_table(table, W, b):
    V, D = table.shape
    H = W.shape[1]
    Wp = jnp.pad(W, ((0, 0), (0, D - H)))
    bp = jnp.pad(b, (0, D - H)).reshape(1, D)
    grid = pl.cdiv(V, _ROW_BLK)
    return pl.pallas_call(
        _proj_body,
        grid=(grid,),
        in_specs=[
            pl.BlockSpec((_ROW_BLK, D), lambda i: (i, 0)),
            pl.BlockSpec((D, D), lambda i: (0, 0)),
            pl.BlockSpec((1, D), lambda i: (0, 0)),
        ],
        out_specs=pl.BlockSpec((_ROW_BLK, D), lambda i: (i, 0)),
        out_shape=jax.ShapeDtypeStruct((V, D), jnp.float32),
    )(table, Wp, bp)


def _make_gather(n, s, D, H):
    assert n % _NW == 0 and s == _CH[0] + _CH[1]
    rpw = n // _NW
    ipw = rpw * s

    mesh = plsc.VectorSubcoreMesh(
        core_axis_name="c", subcore_axis_name="s",
        num_cores=_NC, num_subcores=_NS,
    )

    @functools.partial(
        pl.kernel,
        out_type=jax.ShapeDtypeStruct((n, s, H), jnp.float32),
        mesh=mesh,
        scratch_types=[
            pltpu.VMEM((ipw,), jnp.int32),
            pltpu.VMEM((_CH[0], D), jnp.float32),
            pltpu.VMEM((_CH[1], D), jnp.float32),
            pltpu.VMEM((_CH[0], H), jnp.float32),
            pltpu.VMEM((_CH[1], H), jnp.float32),
            pltpu.SemaphoreType.DMA,
            pltpu.SemaphoreType.DMA,
            pltpu.SemaphoreType.DMA,
            pltpu.SemaphoreType.DMA,
        ],
    )
    def gather(ytab_hbm, idx_hbm, out_hbm, idx_v, ga, gb, pa, pb,
               gsa, gsb, wsa, wsb):
        wid = lax.axis_index("s") * _NC + lax.axis_index("c")
        row0 = wid * rpw
        pltpu.sync_copy(idx_hbm.at[pl.ds(wid * ipw, ipw)], idx_v)

        bufg = (ga, gb)
        bufp = (pa, pb)
        gsem = (gsa, gsb)
        wsem = (wsa, wsb)

        def fire_gather(r, h):
            pltpu.async_copy(
                ytab_hbm.at[idx_v.at[pl.ds(r * s + _OFF[h], _CH[h])]],
                bufg[h], gsem[h])

        def wb_copy(r, h):
            return pltpu.make_async_copy(
                bufp[h],
                out_hbm.at[row0 + r, pl.ds(_OFF[h], _CH[h])],
                wsem[h])

        fire_gather(0, 0)
        fire_gather(0, 1)

        def body(r, carry):
            for h in (0, 1):
                pltpu.make_async_copy(
                    ytab_hbm.at[idx_v.at[pl.ds(r * s + _OFF[h], _CH[h])]],
                    bufg[h], gsem[h]).wait()

                @pl.when(r > 0)
                def _():
                    wb_copy(r - 1, h).wait()

                def compact(r8, c2):
                    for k in range(8):
                        for c in range(H // 16):
                            sl = pl.ds(c * 16, 16)
                            bufp[h][r8 * 8 + k, sl] = bufg[h][r8 * 8 + k, sl]
                    return c2

                lax.fori_loop(0, _CH[h] // 8, compact, 0)
                wb_copy(r, h).start()

                @pl.when(r + 1 < rpw)
                def _():
                    fire_gather(r + 1, h)
            return carry

        lax.fori_loop(0, rpw, body, 0)
        wb_copy(rpw - 1, 0).wait()
        wb_copy(rpw - 1, 1).wait()

    return gather


def kernel(pcp_batch, pcp_masks, table, W, b):
    n, s = pcp_batch.shape
    H = W.shape[1]
    ytab = _project_table(table, W, b)
    idx = pcp_batch.reshape(-1).astype(jnp.int32)
    y = _make_gather(n, s, table.shape[1], H)(ytab, idx)
    return y, pcp_masks
